# baseline (speedup 1.0000x reference)
import functools

import jax
import jax.numpy as jnp
from jax import lax
from jax.experimental import pallas as pl
from jax.experimental.pallas import tpu as pltpu
from jax.experimental.pallas import tpu_sc as plsc

VOCAB = 1000
SEQ = 200
BATCH = 4096
NCLS = 4
TROWS = 1008
PITCH = 209


def _proj_body(emb_ref, w_ref, b_ref, out_ref):
    t = lax.dot_general(
        emb_ref[:], w_ref[:],
        dimension_numbers=(((1,), (1,)), ((), ())),
        preferred_element_type=jnp.float32,
    )
    out_ref[0:VOCAB, :] = t * (1.0 / SEQ)
    out_ref[VOCAB:TROWS, :] = b_ref[:]


def _project_table(emb_table, W, b):
    bpad = jnp.zeros((TROWS - VOCAB, NCLS), jnp.float32).at[0].set(b)
    return pl.pallas_call(
        _proj_body,
        out_shape=jax.ShapeDtypeStruct((TROWS, NCLS), jnp.float32),
    )(emb_table, W, bpad)


def _pack_pairs(T):
    tb = T.astype(jnp.bfloat16).reshape(TROWS, 2, 2)
    w = lax.bitcast_convert_type(tb, jnp.int32)
    return jnp.concatenate([w[:, 0], w[:, 1]])


def _pool_softmax(t2, x):
    info = plsc.get_sparse_core_info()
    nc, ns, L = info.num_cores, info.num_subcores, info.num_lanes
    nw = nc * ns
    b_per_w = BATCH // nw
    groups = b_per_w // L
    mesh = plsc.VectorSubcoreMesh(core_axis_name="c", subcore_axis_name="s")

    def unpack_f32(acc):
        u = plsc.bitcast(acc, jnp.int32)
        himask = jnp.full((L,), -65536, jnp.int32)
        lo = plsc.bitcast(lax.shift_left(u, 16), jnp.float32)
        hi = plsc.bitcast(jnp.bitwise_and(u, himask), jnp.float32)
        return lo, hi

    @functools.partial(
        pl.kernel,
        mesh=mesh,
        compiler_params=pltpu.CompilerParams(
            needs_layout_passes=False, use_tc_tiling_on_sc=False),
        out_type=jax.ShapeDtypeStruct((BATCH, NCLS), jnp.float32),
        scratch_types=[
            pltpu.VMEM((2 * TROWS,), jnp.int32),
            pltpu.VMEM((2, L, PITCH), jnp.int32),
            pltpu.VMEM((b_per_w, NCLS), jnp.float32),
            pltpu.SemaphoreType.DMA,
            pltpu.SemaphoreType.DMA,
        ],
    )
    def k(t_hbm, x_hbm, out_hbm, t_v, xg, o_v, sem0, sem1):
        wid = lax.axis_index("s") * nc + lax.axis_index("c")
        base = wid * b_per_w
        sems = (sem0, sem1)

        def start_fetch(g):
            return pltpu.async_copy(
                x_hbm.at[pl.ds(base + g * L, L)],
                xg.at[g % 2, :, pl.ds(0, SEQ)],
                sems[g % 2],
            )

        pending = start_fetch(0)
        pltpu.sync_copy(t_hbm, t_v)

        iota = lax.iota(jnp.int32, L)
        bias_lo = jnp.full((L,), VOCAB, jnp.int32)
        bias_hi = jnp.full((L,), TROWS + VOCAB, jnp.int32)
        cols = [jnp.full((L,), c, jnp.int32) for c in range(NCLS)]

        for g in range(groups):
            pending.wait()
            if g + 1 < groups:
                pending = start_fetch(g + 1)

            buf = jnp.full((L,), g % 2, jnp.int32)
            acc_a = plsc.bitcast(plsc.load_gather(t_v, [bias_lo]),
                                 jnp.bfloat16)
            acc_b = plsc.bitcast(plsc.load_gather(t_v, [bias_hi]),
                                 jnp.bfloat16)

            def step(l, accs):
                aa, ab = accs
                lv = jnp.broadcast_to(l, (L,)).astype(jnp.int32)
                idx = plsc.load_gather(xg, [buf, iota, lv])
                g0 = plsc.load_gather(t_v, [idx])
                g1 = plsc.load_gather(t_v, [idx + TROWS])
                return (aa + plsc.bitcast(g0, jnp.bfloat16),
                        ab + plsc.bitcast(g1, jnp.bfloat16))

            acc_a, acc_b = lax.fori_loop(0, SEQ, step, (acc_a, acc_b),
                                         unroll=4)

            a0, a1 = unpack_f32(acc_a)
            a2, a3 = unpack_f32(acc_b)
            m = jnp.maximum(jnp.maximum(a0, a1), jnp.maximum(a2, a3))
            e0 = jnp.exp(a0 - m)
            e1 = jnp.exp(a1 - m)
            e2 = jnp.exp(a2 - m)
            e3 = jnp.exp(a3 - m)
            s = (e0 + e1) + (e2 + e3)
            rows = g * L + iota
            for c, ec in enumerate((e0, e1, e2, e3)):
                plsc.store_scatter(o_v, [rows, cols[c]], ec / s)

        pltpu.sync_copy(o_v, out_hbm.at[pl.ds(base, b_per_w)])

    return k(t2, x)


def kernel(x, emb_table, W, b):
    t2 = _pack_pairs(_project_table(emb_table, W, b))
    return _pool_softmax(t2, x)
